# Initial kernel scaffold; baseline (speedup 1.0000x reference)
import jax
import jax.numpy as jnp
from jax.experimental import pallas as pl


def kernel(src_0, src_1, seg_0, seg_1, W0, gamma0, beta0, W1, pos_table, seg_table, gamma1, beta1):
    raise NotImplementedError("write your pallas kernel here")


# speedup vs baseline: 6.4451x; 6.4451x over previous
import functools

import jax
import jax.numpy as jnp
from jax import lax
from jax.experimental import pallas as pl
from jax.experimental.pallas import tpu as pltpu
from jax.experimental.pallas import tpu_sc as plsc

VOCAB = 100000
D = 128
B = 1024
S = 200
N = B * S
EPS = 1e-6

NUM_CORES = 2
NUM_SUBCORES = 16
NW = NUM_CORES * NUM_SUBCORES
ROWS_PER_W = N // NW
CHUNK = 128
NCHUNK = ROWS_PER_W // CHUNK


def _dual_gather(src0_flat, src1_flat, W0, W1):
    mesh = plsc.VectorSubcoreMesh(core_axis_name="c", subcore_axis_name="s")

    @functools.partial(
        pl.kernel,
        mesh=mesh,
        out_type=[
            jax.ShapeDtypeStruct((N, D), jnp.float32),
            jax.ShapeDtypeStruct((N, D), jnp.float32),
        ],
        scratch_types=[
            pltpu.VMEM((ROWS_PER_W,), jnp.int32),
            pltpu.VMEM((ROWS_PER_W,), jnp.int32),
            pltpu.VMEM((CHUNK, D), jnp.float32),
            pltpu.VMEM((CHUNK, D), jnp.float32),
            pltpu.SemaphoreType.DMA,
            pltpu.SemaphoreType.DMA,
        ],
    )
    def body(w0_hbm, w1_hbm, i0_hbm, i1_hbm, o0_hbm, o1_hbm,
             idx0_v, idx1_v, rows0_v, rows1_v, sem0, sem1):
        wid = lax.axis_index("s") * NUM_CORES + lax.axis_index("c")
        base = wid * ROWS_PER_W
        pltpu.sync_copy(i0_hbm.at[pl.ds(base, ROWS_PER_W)], idx0_v)
        pltpu.sync_copy(i1_hbm.at[pl.ds(base, ROWS_PER_W)], idx1_v)

        def step(i, _):
            off = i * CHUNK
            cp0 = pltpu.async_copy(
                w0_hbm.at[idx0_v.at[pl.ds(off, CHUNK)]], rows0_v, sem0)
            cp1 = pltpu.async_copy(
                w1_hbm.at[idx1_v.at[pl.ds(off, CHUNK)]], rows1_v, sem1)
            cp0.wait()
            pltpu.sync_copy(rows0_v, o0_hbm.at[pl.ds(base + off, CHUNK)])
            cp1.wait()
            pltpu.sync_copy(rows1_v, o1_hbm.at[pl.ds(base + off, CHUNK)])
            return 0

        lax.fori_loop(0, NCHUNK, step, 0)

    return body(W0, W1, src0_flat, src1_flat)


BB = 16


def _ln_kernel(raw0_ref, raw1_ref, seg_ref, pos_ref, segtab_ref,
               g0_ref, b0_ref, g1_ref, b1_ref, o0_ref, o1_ref):
    g0 = g0_ref[...]
    b0 = b0_ref[...]
    g1 = g1_ref[...]
    b1 = b1_ref[...]

    def ln(x, g, bta):
        mean = jnp.mean(x, axis=-1, keepdims=True)
        var = jnp.sum((x - mean) ** 2, axis=-1, keepdims=True) / (D - 1)
        std = jnp.sqrt(var)
        return g * (x - mean) / (std + EPS) + bta

    x0 = raw0_ref[...]
    o0_ref[...] = ln(x0, g0, b0)

    seg = seg_ref[...]
    st = segtab_ref[...]
    seg3 = seg[..., None]
    segemb = (jnp.where(seg3 == 0, 1.0, 0.0) * st[0]
              + jnp.where(seg3 == 1, 1.0, 0.0) * st[1]
              + jnp.where(seg3 == 2, 1.0, 0.0) * st[2])
    x1 = raw1_ref[...] + pos_ref[...][None, :, :] + segemb
    o1_ref[...] = ln(x1, g1, b1)


def _ln_call(raw0, raw1, seg_1, pos_slice, seg_table, gamma0, beta0, gamma1, beta1):
    grid = (B // BB,)
    return pl.pallas_call(
        _ln_kernel,
        grid=grid,
        in_specs=[
            pl.BlockSpec((BB, S, D), lambda i: (i, 0, 0)),
            pl.BlockSpec((BB, S, D), lambda i: (i, 0, 0)),
            pl.BlockSpec((BB, S), lambda i: (i, 0)),
            pl.BlockSpec((S, D), lambda i: (0, 0)),
            pl.BlockSpec((3, D), lambda i: (0, 0)),
            pl.BlockSpec((1, D), lambda i: (0, 0)),
            pl.BlockSpec((1, D), lambda i: (0, 0)),
            pl.BlockSpec((1, D), lambda i: (0, 0)),
            pl.BlockSpec((1, D), lambda i: (0, 0)),
        ],
        out_specs=[
            pl.BlockSpec((BB, S, D), lambda i: (i, 0, 0)),
            pl.BlockSpec((BB, S, D), lambda i: (i, 0, 0)),
        ],
        out_shape=[
            jax.ShapeDtypeStruct((B, S, D), jnp.float32),
            jax.ShapeDtypeStruct((B, S, D), jnp.float32),
        ],
    )(raw0, raw1, seg_1, pos_slice, seg_table, gamma0, beta0, gamma1, beta1)


def kernel(src_0, src_1, seg_0, seg_1, W0, gamma0, beta0, W1, pos_table,
           seg_table, gamma1, beta1):
    src0_flat = src_0.reshape(N).astype(jnp.int32)
    src1_flat = src_1.reshape(N).astype(jnp.int32)
    raw0, raw1 = _dual_gather(src0_flat, src1_flat, W0, W1)
    raw0 = raw0.reshape(B, S, D)
    raw1 = raw1.reshape(B, S, D)
    e0, e1 = _ln_call(
        raw0, raw1, seg_1.astype(jnp.int32), pos_table[:S], seg_table,
        gamma0.reshape(1, D), beta0.reshape(1, D),
        gamma1.reshape(1, D), beta1.reshape(1, D))
    return (e0, e1)
